# Initial kernel scaffold; baseline (speedup 1.0000x reference)
import jax
import jax.numpy as jnp
from jax.experimental import pallas as pl


def kernel(hidden_states, router_weight, expert_up, expert_down, shared_up, shared_down, attn_metadata):
    raise NotImplementedError("write your pallas kernel here")


# speedup vs baseline: 1.1784x; 1.1784x over previous
import functools

import jax
import jax.numpy as jnp
import numpy as np
from jax.experimental import pallas as pl
from jax.experimental.pallas import tpu as pltpu

E = 8
N_GROUP = 4
TOP_K = 2
TOPK_GROUP = 2
HIDDEN = 1024
MOE_INTER = 512
SCALING = 2.5

TOKEN_TILE = 256


_PGROUP = np.kron(np.eye(N_GROUP, dtype=np.float32),
                  np.ones((E // N_GROUP, E // N_GROUP), dtype=np.float32))


def _moe_kernel(xb_ref, rw_ref, pg_ref, up_ref, dn_ref, su_ref,
                sd_ref, out_ref):
    xb = xb_ref[...]

    logits = jax.lax.dot_general(
        xb, rw_ref[...], (((1,), (1,)), ((), ())),
        preferred_element_type=jnp.float32)
    s = jax.nn.sigmoid(logits)
    T = s.shape[0]

    lane = jax.lax.broadcasted_iota(jnp.int32, (T, E), 1)
    gidx = lane // (E // N_GROUP)

    gs8 = jax.lax.dot_general(
        s, pg_ref[...], (((1,), (0,)), ((), ())),
        preferred_element_type=jnp.float32,
        precision=jax.lax.Precision.HIGHEST)

    gcnt = jnp.zeros((T, E), jnp.float32)
    for g in range(N_GROUP):
        col = gs8[:, g * 2:g * 2 + 1]
        beats = (col > gs8) | ((col == gs8) & (g < gidx))
        gcnt = gcnt + jnp.where(beats, 1.0, 0.0)
    gmask = jnp.where(gcnt < TOPK_GROUP, 1.0, 0.0)

    m = gmask * s

    ecnt = jnp.zeros((T, E), jnp.float32)
    for e in range(E):
        col = m[:, e:e + 1]
        beats = (col > m) | ((col == m) & (e < lane))
        ecnt = ecnt + jnp.where(beats, 1.0, 0.0)
    sel = jnp.where(ecnt < TOP_K, 1.0, 0.0)

    picked = sel * s
    denom = jnp.sum(picked, axis=1, keepdims=True) + 1e-20
    w_e = picked * (SCALING / denom)

    acc = None
    for e in range(E):
        h = jax.lax.dot_general(
            xb, up_ref[e], (((1,), (1,)), ((), ())),
            preferred_element_type=jnp.float32)
        h = jnp.maximum(h, 0.0).astype(jnp.bfloat16)
        o = jax.lax.dot_general(
            h, dn_ref[e], (((1,), (1,)), ((), ())),
            preferred_element_type=jnp.float32)
        o = o * w_e[:, e:e + 1]
        acc = o if acc is None else acc + o

    sh = jax.lax.dot_general(
        xb, su_ref[...], (((1,), (1,)), ((), ())),
        preferred_element_type=jnp.float32)
    sh = jnp.maximum(sh, 0.0).astype(jnp.bfloat16)
    acc = acc + jax.lax.dot_general(
        sh, sd_ref[...], (((1,), (1,)), ((), ())),
        preferred_element_type=jnp.float32)

    out_ref[...] = acc


@functools.partial(jax.jit, static_argnames=())
def kernel(hidden_states, router_weight, expert_up, expert_down,
           shared_up, shared_down, attn_metadata):
    orig_shape = hidden_states.shape
    x = hidden_states.reshape(-1, HIDDEN)
    T = x.shape[0]
    xb = x.astype(jnp.bfloat16)
    up_b = expert_up.astype(jnp.bfloat16)
    dn_b = expert_down.astype(jnp.bfloat16)
    su_b = shared_up.astype(jnp.bfloat16)
    sd_b = shared_down.astype(jnp.bfloat16)

    grid = (T // TOKEN_TILE,)
    out = pl.pallas_call(
        _moe_kernel,
        grid=grid,
        in_specs=[
            pl.BlockSpec((TOKEN_TILE, HIDDEN), lambda i: (i, 0)),
            pl.BlockSpec((E, HIDDEN), lambda i: (0, 0)),
            pl.BlockSpec((E, E), lambda i: (0, 0)),
            pl.BlockSpec((E, MOE_INTER, HIDDEN), lambda i: (0, 0, 0)),
            pl.BlockSpec((E, HIDDEN, MOE_INTER), lambda i: (0, 0, 0)),
            pl.BlockSpec((HIDDEN, HIDDEN), lambda i: (0, 0)),
            pl.BlockSpec((HIDDEN, HIDDEN), lambda i: (0, 0)),
        ],
        out_specs=pl.BlockSpec((TOKEN_TILE, HIDDEN), lambda i: (i, 0)),
        out_shape=jax.ShapeDtypeStruct((T, HIDDEN), jnp.float32),
        compiler_params=pltpu.CompilerParams(
            dimension_semantics=("arbitrary",),
        ),
    )(xb, router_weight.astype(jnp.bfloat16), jnp.asarray(_PGROUP),
      up_b, dn_b, su_b, sd_b)
    return out.reshape(orig_shape)


# speedup vs baseline: 1.3089x; 1.1107x over previous
import functools

import jax
import jax.numpy as jnp
import numpy as np
from jax.experimental import pallas as pl
from jax.experimental.pallas import tpu as pltpu

E = 8
N_GROUP = 4
TOP_K = 2
TOPK_GROUP = 2
HIDDEN = 1024
MOE_INTER = 512
SCALING = 2.5

TOKEN_TILE = 512


_PGROUP = np.kron(np.eye(N_GROUP, dtype=np.float32),
                  np.ones((E // N_GROUP, E // N_GROUP), dtype=np.float32))


def _moe_kernel(xb_ref, rw_ref, pg_ref, up_ref, dn_ref, su_ref,
                sd_ref, out_ref):
    xb = xb_ref[...]

    logits = jax.lax.dot_general(
        xb, rw_ref[...], (((1,), (1,)), ((), ())),
        preferred_element_type=jnp.float32)
    s = jax.nn.sigmoid(logits)
    T = s.shape[0]

    lane = jax.lax.broadcasted_iota(jnp.int32, (T, E), 1)
    gidx = lane // (E // N_GROUP)

    gs8 = jax.lax.dot_general(
        s, pg_ref[...], (((1,), (0,)), ((), ())),
        preferred_element_type=jnp.float32,
        precision=jax.lax.Precision.HIGHEST)

    gcnt = jnp.zeros((T, E), jnp.float32)
    for g in range(N_GROUP):
        col = gs8[:, g * 2:g * 2 + 1]
        beats = (col > gs8) | ((col == gs8) & (g < gidx))
        gcnt = gcnt + jnp.where(beats, 1.0, 0.0)
    gmask = jnp.where(gcnt < TOPK_GROUP, 1.0, 0.0)

    m = gmask * s

    ecnt = jnp.zeros((T, E), jnp.float32)
    for e in range(E):
        col = m[:, e:e + 1]
        beats = (col > m) | ((col == m) & (e < lane))
        ecnt = ecnt + jnp.where(beats, 1.0, 0.0)
    sel = jnp.where(ecnt < TOP_K, 1.0, 0.0)

    picked = sel * s
    denom = jnp.sum(picked, axis=1, keepdims=True) + 1e-20
    w_e = picked * (SCALING / denom)

    acc = None
    for e in range(E):
        h = jax.lax.dot_general(
            xb, up_ref[e], (((1,), (1,)), ((), ())),
            preferred_element_type=jnp.float32)
        h = jnp.maximum(h, 0.0).astype(jnp.bfloat16)
        o = jax.lax.dot_general(
            h, dn_ref[e], (((1,), (1,)), ((), ())),
            preferred_element_type=jnp.float32)
        o = o * w_e[:, e:e + 1]
        acc = o if acc is None else acc + o

    sh = jax.lax.dot_general(
        xb, su_ref[...], (((1,), (1,)), ((), ())),
        preferred_element_type=jnp.float32)
    sh = jnp.maximum(sh, 0.0).astype(jnp.bfloat16)
    acc = acc + jax.lax.dot_general(
        sh, sd_ref[...], (((1,), (1,)), ((), ())),
        preferred_element_type=jnp.float32)

    out_ref[...] = acc


@functools.partial(jax.jit, static_argnames=())
def kernel(hidden_states, router_weight, expert_up, expert_down,
           shared_up, shared_down, attn_metadata):
    orig_shape = hidden_states.shape
    x = hidden_states.reshape(-1, HIDDEN)
    T = x.shape[0]
    xb = x.astype(jnp.bfloat16)
    up_b = expert_up.astype(jnp.bfloat16)
    dn_b = expert_down.astype(jnp.bfloat16)
    su_b = shared_up.astype(jnp.bfloat16)
    sd_b = shared_down.astype(jnp.bfloat16)

    grid = (T // TOKEN_TILE,)
    out = pl.pallas_call(
        _moe_kernel,
        grid=grid,
        in_specs=[
            pl.BlockSpec((TOKEN_TILE, HIDDEN), lambda i: (i, 0)),
            pl.BlockSpec((E, HIDDEN), lambda i: (0, 0)),
            pl.BlockSpec((E, E), lambda i: (0, 0)),
            pl.BlockSpec((E, MOE_INTER, HIDDEN), lambda i: (0, 0, 0)),
            pl.BlockSpec((E, HIDDEN, MOE_INTER), lambda i: (0, 0, 0)),
            pl.BlockSpec((HIDDEN, HIDDEN), lambda i: (0, 0)),
            pl.BlockSpec((HIDDEN, HIDDEN), lambda i: (0, 0)),
        ],
        out_specs=pl.BlockSpec((TOKEN_TILE, HIDDEN), lambda i: (i, 0)),
        out_shape=jax.ShapeDtypeStruct((T, HIDDEN), jnp.float32),
        compiler_params=pltpu.CompilerParams(
            dimension_semantics=("parallel",),
        ),
    )(xb, router_weight.astype(jnp.bfloat16), jnp.asarray(_PGROUP),
      up_b, dn_b, su_b, sd_b)
    return out.reshape(orig_shape)


# speedup vs baseline: 1.3168x; 1.0060x over previous
import functools

import jax
import jax.numpy as jnp
import numpy as np
from jax.experimental import pallas as pl
from jax.experimental.pallas import tpu as pltpu

E = 8
N_GROUP = 4
TOP_K = 2
TOPK_GROUP = 2
HIDDEN = 1024
MOE_INTER = 512
SCALING = 2.5

TOKEN_TILE = 1024


_PGROUP = np.kron(np.eye(N_GROUP, dtype=np.float32),
                  np.ones((E // N_GROUP, E // N_GROUP), dtype=np.float32))


def _moe_kernel(xb_ref, rw_ref, pg_ref, up_ref, dn_ref, su_ref,
                sd_ref, out_ref):
    xb = xb_ref[...]

    logits = jax.lax.dot_general(
        xb, rw_ref[...], (((1,), (1,)), ((), ())),
        preferred_element_type=jnp.float32)
    s = jax.nn.sigmoid(logits)
    T = s.shape[0]

    lane = jax.lax.broadcasted_iota(jnp.int32, (T, E), 1)
    gidx = lane // (E // N_GROUP)

    gs8 = jax.lax.dot_general(
        s, pg_ref[...], (((1,), (0,)), ((), ())),
        preferred_element_type=jnp.float32,
        precision=jax.lax.Precision.HIGHEST)

    gcnt = jnp.zeros((T, E), jnp.float32)
    for g in range(N_GROUP):
        col = gs8[:, g * 2:g * 2 + 1]
        beats = (col > gs8) | ((col == gs8) & (g < gidx))
        gcnt = gcnt + jnp.where(beats, 1.0, 0.0)
    gmask = jnp.where(gcnt < TOPK_GROUP, 1.0, 0.0)

    m = gmask * s

    ecnt = jnp.zeros((T, E), jnp.float32)
    for e in range(E):
        col = m[:, e:e + 1]
        beats = (col > m) | ((col == m) & (e < lane))
        ecnt = ecnt + jnp.where(beats, 1.0, 0.0)
    sel = jnp.where(ecnt < TOP_K, 1.0, 0.0)

    picked = sel * s
    denom = jnp.sum(picked, axis=1, keepdims=True) + 1e-20
    w_e = picked * (SCALING / denom)

    acc = None
    for e in range(E):
        h = jax.lax.dot_general(
            xb, up_ref[e], (((1,), (1,)), ((), ())),
            preferred_element_type=jnp.float32)
        h = jnp.maximum(h, 0.0).astype(jnp.bfloat16)
        o = jax.lax.dot_general(
            h, dn_ref[e], (((1,), (1,)), ((), ())),
            preferred_element_type=jnp.float32)
        o = o * w_e[:, e:e + 1]
        acc = o if acc is None else acc + o

    sh = jax.lax.dot_general(
        xb, su_ref[...], (((1,), (1,)), ((), ())),
        preferred_element_type=jnp.float32)
    sh = jnp.maximum(sh, 0.0).astype(jnp.bfloat16)
    acc = acc + jax.lax.dot_general(
        sh, sd_ref[...], (((1,), (1,)), ((), ())),
        preferred_element_type=jnp.float32)

    out_ref[...] = acc


@functools.partial(jax.jit, static_argnames=())
def kernel(hidden_states, router_weight, expert_up, expert_down,
           shared_up, shared_down, attn_metadata):
    orig_shape = hidden_states.shape
    x = hidden_states.reshape(-1, HIDDEN)
    T = x.shape[0]
    xb = x.astype(jnp.bfloat16)
    up_b = expert_up.astype(jnp.bfloat16)
    dn_b = expert_down.astype(jnp.bfloat16)
    su_b = shared_up.astype(jnp.bfloat16)
    sd_b = shared_down.astype(jnp.bfloat16)

    grid = (T // TOKEN_TILE,)
    out = pl.pallas_call(
        _moe_kernel,
        grid=grid,
        in_specs=[
            pl.BlockSpec((TOKEN_TILE, HIDDEN), lambda i: (i, 0)),
            pl.BlockSpec((E, HIDDEN), lambda i: (0, 0)),
            pl.BlockSpec((E, E), lambda i: (0, 0)),
            pl.BlockSpec((E, MOE_INTER, HIDDEN), lambda i: (0, 0, 0)),
            pl.BlockSpec((E, HIDDEN, MOE_INTER), lambda i: (0, 0, 0)),
            pl.BlockSpec((HIDDEN, HIDDEN), lambda i: (0, 0)),
            pl.BlockSpec((HIDDEN, HIDDEN), lambda i: (0, 0)),
        ],
        out_specs=pl.BlockSpec((TOKEN_TILE, HIDDEN), lambda i: (i, 0)),
        out_shape=jax.ShapeDtypeStruct((T, HIDDEN), jnp.float32),
        compiler_params=pltpu.CompilerParams(
            dimension_semantics=("parallel",),
        ),
    )(xb, router_weight.astype(jnp.bfloat16), jnp.asarray(_PGROUP),
      up_b, dn_b, su_b, sd_b)
    return out.reshape(orig_shape)


# speedup vs baseline: 1.5220x; 1.1558x over previous
import functools

import jax
import jax.numpy as jnp
import numpy as np
from jax.experimental import pallas as pl
from jax.experimental.pallas import tpu as pltpu

E = 8
N_GROUP = 4
TOP_K = 2
TOPK_GROUP = 2
HIDDEN = 1024
MOE_INTER = 512
SCALING = 2.5

_PGROUP = np.kron(np.eye(N_GROUP, dtype=np.float32),
                  np.ones((E // N_GROUP, E // N_GROUP), dtype=np.float32))


def _moe_kernel(x_ref, rw_ref, pg_ref, up_ref, dn_ref, su_ref, sd_ref,
                out_ref, xb_s, we_s):
    i = pl.program_id(0)

    @pl.when(i == 0)
    def _prologue():
        xb = x_ref[...].astype(jnp.bfloat16)
        xb_s[...] = xb

        logits = jax.lax.dot_general(
            xb, rw_ref[...].astype(jnp.bfloat16), (((1,), (1,)), ((), ())),
            preferred_element_type=jnp.float32)
        s = jax.nn.sigmoid(logits)
        T = s.shape[0]

        lane = jax.lax.broadcasted_iota(jnp.int32, (T, E), 1)
        gidx = lane // (E // N_GROUP)

        gs8 = jax.lax.dot_general(
            s, pg_ref[...], (((1,), (0,)), ((), ())),
            preferred_element_type=jnp.float32,
            precision=jax.lax.Precision.HIGHEST)

        gcnt = jnp.zeros((T, E), jnp.float32)
        for g in range(N_GROUP):
            col = gs8[:, g * 2:g * 2 + 1]
            beats = (col > gs8) | ((col == gs8) & (g < gidx))
            gcnt = gcnt + jnp.where(beats, 1.0, 0.0)
        gmask = jnp.where(gcnt < TOPK_GROUP, 1.0, 0.0)

        m = gmask * s

        ecnt = jnp.zeros((T, E), jnp.float32)
        for e in range(E):
            col = m[:, e:e + 1]
            beats = (col > m) | ((col == m) & (e < lane))
            ecnt = ecnt + jnp.where(beats, 1.0, 0.0)
        sel = jnp.where(ecnt < TOP_K, 1.0, 0.0)

        picked = sel * s
        denom = jnp.sum(picked, axis=1, keepdims=True) + 1e-20
        we_s[...] = picked * (SCALING / denom)

    @pl.when(i < E)
    def _expert():
        xb = xb_s[...]
        up_b = up_ref[0].astype(jnp.bfloat16)
        dn_b = dn_ref[0].astype(jnp.bfloat16)
        h = jax.lax.dot_general(
            xb, up_b, (((1,), (1,)), ((), ())),
            preferred_element_type=jnp.float32)
        h = jnp.maximum(h, 0.0).astype(jnp.bfloat16)
        o = jax.lax.dot_general(
            h, dn_b, (((1,), (1,)), ((), ())),
            preferred_element_type=jnp.float32)

        w_all = we_s[...]
        lane = jax.lax.broadcasted_iota(jnp.int32, w_all.shape, 1)
        w_col = jnp.sum(jnp.where(lane == i, w_all, 0.0), axis=1,
                        keepdims=True)
        contrib = o * w_col

        @pl.when(i == 0)
        def _init():
            out_ref[...] = contrib

        @pl.when(i > 0)
        def _acc():
            out_ref[...] = out_ref[...] + contrib

    @pl.when(i == E)
    def _shared():
        xb = xb_s[...]
        sh = jax.lax.dot_general(
            xb, su_ref[...].astype(jnp.bfloat16), (((1,), (1,)), ((), ())),
            preferred_element_type=jnp.float32)
        sh = jnp.maximum(sh, 0.0).astype(jnp.bfloat16)
        out_ref[...] = out_ref[...] + jax.lax.dot_general(
            sh, sd_ref[...].astype(jnp.bfloat16), (((1,), (1,)), ((), ())),
            preferred_element_type=jnp.float32)


@functools.partial(jax.jit, static_argnames=())
def kernel(hidden_states, router_weight, expert_up, expert_down,
           shared_up, shared_down, attn_metadata):
    orig_shape = hidden_states.shape
    x = hidden_states.reshape(-1, HIDDEN)
    T = x.shape[0]

    out = pl.pallas_call(
        _moe_kernel,
        grid=(E + 1,),
        in_specs=[
            pl.BlockSpec((T, HIDDEN), lambda i: (0, 0)),
            pl.BlockSpec((E, HIDDEN), lambda i: (0, 0)),
            pl.BlockSpec((E, E), lambda i: (0, 0)),
            pl.BlockSpec((1, MOE_INTER, HIDDEN),
                         lambda i: (jnp.minimum(i, E - 1), 0, 0)),
            pl.BlockSpec((1, HIDDEN, MOE_INTER),
                         lambda i: (jnp.minimum(i, E - 1), 0, 0)),
            pl.BlockSpec((HIDDEN, HIDDEN), lambda i: (0, 0)),
            pl.BlockSpec((HIDDEN, HIDDEN), lambda i: (0, 0)),
        ],
        out_specs=pl.BlockSpec((T, HIDDEN), lambda i: (0, 0)),
        out_shape=jax.ShapeDtypeStruct((T, HIDDEN), jnp.float32),
        scratch_shapes=[
            pltpu.VMEM((T, HIDDEN), jnp.bfloat16),
            pltpu.VMEM((T, E), jnp.float32),
        ],
        compiler_params=pltpu.CompilerParams(
            dimension_semantics=("arbitrary",),
        ),
    )(x, router_weight, jnp.asarray(_PGROUP), expert_up, expert_down,
      shared_up, shared_down)
    return out.reshape(orig_shape)


# speedup vs baseline: 1.5231x; 1.0008x over previous
import functools

import jax
import jax.numpy as jnp
import numpy as np
from jax.experimental import pallas as pl
from jax.experimental.pallas import tpu as pltpu

E = 8
N_GROUP = 4
TOP_K = 2
TOPK_GROUP = 2
HIDDEN = 1024
MOE_INTER = 512
SCALING = 2.5
EPG = 1

_PGROUP = np.kron(np.eye(N_GROUP, dtype=np.float32),
                  np.ones((E // N_GROUP, E // N_GROUP), dtype=np.float32))


def _moe_kernel(x_ref, rw_ref, pg_ref, up_ref, dn_ref, su_ref, sd_ref,
                out_ref, xb_s, we_s):
    i = pl.program_id(0)

    @pl.when(i == 0)
    def _prologue():
        xb = x_ref[...].astype(jnp.bfloat16)
        xb_s[...] = xb

        logits = jax.lax.dot_general(
            xb, rw_ref[...].astype(jnp.bfloat16), (((1,), (1,)), ((), ())),
            preferred_element_type=jnp.float32)
        s = jax.nn.sigmoid(logits)
        T = s.shape[0]

        lane = jax.lax.broadcasted_iota(jnp.int32, (T, E), 1)
        gidx = lane // (E // N_GROUP)

        gs8 = jax.lax.dot_general(
            s, pg_ref[...], (((1,), (0,)), ((), ())),
            preferred_element_type=jnp.float32,
            precision=jax.lax.Precision.HIGHEST)

        gcnt = jnp.zeros((T, E), jnp.float32)
        for g in range(N_GROUP):
            col = gs8[:, g * 2:g * 2 + 1]
            beats = (col > gs8) | ((col == gs8) & (g < gidx))
            gcnt = gcnt + jnp.where(beats, 1.0, 0.0)
        gmask = jnp.where(gcnt < TOPK_GROUP, 1.0, 0.0)

        m = gmask * s

        ecnt = jnp.zeros((T, E), jnp.float32)
        for e in range(E):
            col = m[:, e:e + 1]
            beats = (col > m) | ((col == m) & (e < lane))
            ecnt = ecnt + jnp.where(beats, 1.0, 0.0)
        sel = jnp.where(ecnt < TOP_K, 1.0, 0.0)

        picked = sel * s
        denom = jnp.sum(picked, axis=1, keepdims=True) + 1e-20
        we_s[...] = picked * (SCALING / denom)

    @pl.when(i < E // EPG)
    def _expert():
        xb = xb_s[...]
        w_all = we_s[...]
        lane = jax.lax.broadcasted_iota(jnp.int32, w_all.shape, 1)
        contrib = None
        for j in range(EPG):
            up_b = up_ref[j].astype(jnp.bfloat16)
            dn_b = dn_ref[j].astype(jnp.bfloat16)
            h = jax.lax.dot_general(
                xb, up_b, (((1,), (1,)), ((), ())),
                preferred_element_type=jnp.float32)
            h = jnp.maximum(h, 0.0).astype(jnp.bfloat16)
            o = jax.lax.dot_general(
                h, dn_b, (((1,), (1,)), ((), ())),
                preferred_element_type=jnp.float32)

            w_col = jnp.sum(
                jnp.where(lane == i * EPG + j, w_all, 0.0), axis=1,
                keepdims=True)
            c = o * w_col
            contrib = c if contrib is None else contrib + c

        @pl.when(i == 0)
        def _init():
            out_ref[...] = contrib

        @pl.when(i > 0)
        def _acc():
            out_ref[...] = out_ref[...] + contrib

    @pl.when(i == E // EPG)
    def _shared():
        xb = xb_s[...]
        sh = jax.lax.dot_general(
            xb, su_ref[...].astype(jnp.bfloat16), (((1,), (1,)), ((), ())),
            preferred_element_type=jnp.float32)
        sh = jnp.maximum(sh, 0.0).astype(jnp.bfloat16)
        out_ref[...] = out_ref[...] + jax.lax.dot_general(
            sh, sd_ref[...].astype(jnp.bfloat16), (((1,), (1,)), ((), ())),
            preferred_element_type=jnp.float32)


@functools.partial(jax.jit, static_argnames=())
def kernel(hidden_states, router_weight, expert_up, expert_down,
           shared_up, shared_down, attn_metadata):
    orig_shape = hidden_states.shape
    x = hidden_states.reshape(-1, HIDDEN)
    T = x.shape[0]

    out = pl.pallas_call(
        _moe_kernel,
        grid=(E // EPG + 1,),
        in_specs=[
            pl.BlockSpec((T, HIDDEN), lambda i: (0, 0)),
            pl.BlockSpec((E, HIDDEN), lambda i: (0, 0)),
            pl.BlockSpec((E, E), lambda i: (0, 0)),
            pl.BlockSpec((EPG, MOE_INTER, HIDDEN),
                         lambda i: (jnp.minimum(i, E // EPG - 1), 0, 0)),
            pl.BlockSpec((EPG, HIDDEN, MOE_INTER),
                         lambda i: (jnp.minimum(i, E // EPG - 1), 0, 0)),
            pl.BlockSpec((HIDDEN, HIDDEN), lambda i: (0, 0)),
            pl.BlockSpec((HIDDEN, HIDDEN), lambda i: (0, 0)),
        ],
        out_specs=pl.BlockSpec((T, HIDDEN), lambda i: (0, 0)),
        out_shape=jax.ShapeDtypeStruct((T, HIDDEN), jnp.float32),
        scratch_shapes=[
            pltpu.VMEM((T, HIDDEN), jnp.bfloat16),
            pltpu.VMEM((T, E), jnp.float32),
        ],
        compiler_params=pltpu.CompilerParams(
            dimension_semantics=("arbitrary",),
        ),
    )(x, router_weight, jnp.asarray(_PGROUP), expert_up, expert_down,
      shared_up, shared_down)
    return out.reshape(orig_shape)


# speedup vs baseline: 1.7360x; 1.1398x over previous
import functools

import jax
import jax.numpy as jnp
import numpy as np
from jax.experimental import pallas as pl
from jax.experimental.pallas import tpu as pltpu

E = 8
N_GROUP = 4
TOP_K = 2
TOPK_GROUP = 2
HIDDEN = 1024
MOE_INTER = 512
SCALING = 2.5
EPG = 2

_PGROUP = np.kron(np.eye(N_GROUP, dtype=np.float32),
                  np.ones((E // N_GROUP, E // N_GROUP), dtype=np.float32))


def _moe_kernel(x_ref, rw_ref, pg_ref, up_ref, dn_ref, su_ref, sd_ref,
                out_ref, xb_s, we_s):
    i = pl.program_id(0)

    @pl.when(i == 0)
    def _prologue():
        xb = x_ref[...].astype(jnp.bfloat16)
        xb_s[...] = xb

        logits = jax.lax.dot_general(
            rw_ref[...].astype(jnp.bfloat16), xb, (((1,), (1,)), ((), ())),
            preferred_element_type=jnp.float32)
        s = jax.nn.sigmoid(logits)
        T = s.shape[1]

        erow = jax.lax.broadcasted_iota(jnp.int32, (E, T), 0)
        gidx = erow // (E // N_GROUP)

        gs8 = jax.lax.dot_general(
            pg_ref[...], s, (((1,), (0,)), ((), ())),
            preferred_element_type=jnp.float32,
            precision=jax.lax.Precision.HIGHEST)

        gcnt = jnp.zeros((E, T), jnp.float32)
        for g in range(N_GROUP):
            row = gs8[g * 2:g * 2 + 1, :]
            beats = (row > gs8) | ((row == gs8) & (g < gidx))
            gcnt = gcnt + jnp.where(beats, 1.0, 0.0)
        gmask = jnp.where(gcnt < TOPK_GROUP, 1.0, 0.0)

        m = gmask * s

        ecnt = jnp.zeros((E, T), jnp.float32)
        for e in range(E):
            row = m[e:e + 1, :]
            beats = (row > m) | ((row == m) & (e < erow))
            ecnt = ecnt + jnp.where(beats, 1.0, 0.0)
        sel = jnp.where(ecnt < TOP_K, 1.0, 0.0)

        picked = sel * s
        denom = jnp.sum(picked, axis=0, keepdims=True) + 1e-20
        we_s[...] = picked * (SCALING / denom)

    @pl.when(i < E // EPG)
    def _expert():
        xb = xb_s[...]
        hs = []
        dns = []
        for j in range(EPG):
            up_b = up_ref[j].astype(jnp.bfloat16)
            h = jax.lax.dot_general(
                xb, up_b, (((1,), (1,)), ((), ())),
                preferred_element_type=jnp.float32)
            h = jnp.maximum(h, 0.0)

            onehot = jnp.where(
                jax.lax.broadcasted_iota(jnp.int32, (E, 1), 0) == i * EPG + j,
                1.0, 0.0)
            w_col = jax.lax.dot_general(
                we_s[...], onehot, (((0,), (0,)), ((), ())),
                preferred_element_type=jnp.float32)

            hs.append((h * w_col).astype(jnp.bfloat16))
            dns.append(dn_ref[j].astype(jnp.bfloat16))

        hcat = jnp.concatenate(hs, axis=1)
        dncat = jnp.concatenate(dns, axis=1)
        contrib = jax.lax.dot_general(
            hcat, dncat, (((1,), (1,)), ((), ())),
            preferred_element_type=jnp.float32)

        @pl.when(i == 0)
        def _init():
            out_ref[...] = contrib

        @pl.when(i > 0)
        def _acc():
            out_ref[...] = out_ref[...] + contrib

    @pl.when(i == E // EPG)
    def _shared():
        xb = xb_s[...]
        sh = jax.lax.dot_general(
            xb, su_ref[...].astype(jnp.bfloat16), (((1,), (1,)), ((), ())),
            preferred_element_type=jnp.float32)
        sh = jnp.maximum(sh, 0.0).astype(jnp.bfloat16)
        out_ref[...] = out_ref[...] + jax.lax.dot_general(
            sh, sd_ref[...].astype(jnp.bfloat16), (((1,), (1,)), ((), ())),
            preferred_element_type=jnp.float32)


@functools.partial(jax.jit, static_argnames=())
def kernel(hidden_states, router_weight, expert_up, expert_down,
           shared_up, shared_down, attn_metadata):
    orig_shape = hidden_states.shape
    x = hidden_states.reshape(-1, HIDDEN)
    T = x.shape[0]

    out = pl.pallas_call(
        _moe_kernel,
        grid=(E // EPG + 1,),
        in_specs=[
            pl.BlockSpec((T, HIDDEN), lambda i: (0, 0)),
            pl.BlockSpec((E, HIDDEN), lambda i: (0, 0)),
            pl.BlockSpec((E, E), lambda i: (0, 0)),
            pl.BlockSpec((EPG, MOE_INTER, HIDDEN),
                         lambda i: (jnp.minimum(i, E // EPG - 1), 0, 0)),
            pl.BlockSpec((EPG, HIDDEN, MOE_INTER),
                         lambda i: (jnp.minimum(i, E // EPG - 1), 0, 0)),
            pl.BlockSpec((HIDDEN, HIDDEN), lambda i: (0, 0)),
            pl.BlockSpec((HIDDEN, HIDDEN), lambda i: (0, 0)),
        ],
        out_specs=pl.BlockSpec((T, HIDDEN), lambda i: (0, 0)),
        out_shape=jax.ShapeDtypeStruct((T, HIDDEN), jnp.float32),
        scratch_shapes=[
            pltpu.VMEM((T, HIDDEN), jnp.bfloat16),
            pltpu.VMEM((E, T), jnp.float32),
        ],
        compiler_params=pltpu.CompilerParams(
            dimension_semantics=("arbitrary",),
        ),
    )(x, router_weight, jnp.asarray(_PGROUP), expert_up, expert_down,
      shared_up, shared_down)
    return out.reshape(orig_shape)
